# baseline (speedup 1.0000x reference)
import dataclasses

import jax
import jax.numpy as jnp
from jax import lax
from jax.experimental import pallas as pl
from jax.experimental.pallas import tpu as pltpu
from jax.experimental.pallas import tpu_sc as plsc

N = 10000
E = 320000
D_IN = 128
D_HID = 128
D_OUT = 16

NPAD = 10240
NC = 2
NS = 16
NW = NC * NS
L = 16
K = 128
CH = 80
KS = 32
NBUF = 4
EPAD = NW * CH * K

_mesh = plsc.VectorSubcoreMesh(core_axis_name="c", subcore_axis_name="s")


def _hist_body(dsti_hbm, out_hbm, dstv, cnt):
    c = lax.axis_index("c")
    s = lax.axis_index("s")
    wid = s * NC + c
    pltpu.sync_copy(dsti_hbm.at[wid], dstv)

    zeros = jnp.zeros((L,), jnp.float32)

    @pl.loop(0, NPAD // L, step=8)
    def _(i):
        for t in range(8):
            cnt[pl.ds((i + t) * L, L)] = zeros

    ones = jnp.ones((L,), jnp.float32)

    @pl.loop(0, CH)
    def _(j):
        for t in range(K // L):
            idx = dstv[j, pl.ds(t * L, L)]
            plsc.addupdate_scatter(cnt, [idx], ones)

    pltpu.sync_copy(cnt, out_hbm.at[wid])


_sc_params = pltpu.CompilerParams()
if "needs_layout_passes" in pltpu.CompilerParams.__dataclass_fields__:
    _sc_params = dataclasses.replace(_sc_params, needs_layout_passes=False)

_hist_call = pl.kernel(
    _hist_body,
    out_type=jax.ShapeDtypeStruct((NW, NPAD), jnp.float32),
    mesh=_mesh,
    compiler_params=_sc_params,
    scratch_types=[
        pltpu.VMEM((CH, K), jnp.int32),
        pltpu.VMEM((NPAD,), jnp.float32),
    ],
)


def _scat_body(table_hbm, srci_hbm, dsti_hbm, zero_hbm, out_hbm,
               srcv, dstv, b0, b1, b2, b3, acc, s0, s1, s2, s3):
    bufs = (b0, b1, b2, b3)
    sems = (s0, s1, s2, s3)
    c = lax.axis_index("c")
    s = lax.axis_index("s")
    wid = s * NC + c
    rpt = NPAD // NS
    base = s * rpt
    pltpu.sync_copy(zero_hbm.at[pl.ds(base, rpt)], acc.at[pl.ds(base, rpt)])
    pltpu.sync_copy(srci_hbm.at[wid], srcv)
    pltpu.sync_copy(dsti_hbm.at[wid], dstv)
    plsc.subcore_barrier()

    nq = K // KS
    nch = CH * nq

    def gath(chunk_row, off, q):
        return pltpu.make_async_copy(
            table_hbm.at[srcv.at[chunk_row, pl.ds(off, KS)]], bufs[q], sems[q])

    for q in range(NBUF - 1):
        gath(0, q * KS, q).start()

    @pl.loop(0, CH)
    def _(j):
        for q in range(NBUF):
            t = q + NBUF - 1
            row_adv, off = divmod(t, nq)

            @pl.when(nq * j + t < nch)
            def _():
                gath(j + row_adv, off * KS, (t % NBUF)).start()

            gath(j, q * KS, q).wait()

    plsc.subcore_barrier()
    pltpu.sync_copy(acc.at[pl.ds(base, rpt)], out_hbm.at[c, pl.ds(base, rpt)])


_scat = pl.kernel(
    _scat_body,
    out_type=jax.ShapeDtypeStruct((NC, NPAD, D_HID), jnp.float32),
    mesh=_mesh,
    scratch_types=[
        pltpu.VMEM((CH, K), jnp.int32),
        pltpu.VMEM((CH, K), jnp.int32),
        pltpu.VMEM((KS, D_HID), jnp.float32),
        pltpu.VMEM((KS, D_HID), jnp.float32),
        pltpu.VMEM((KS, D_HID), jnp.float32),
        pltpu.VMEM((KS, D_HID), jnp.float32),
        pltpu.VMEM_SHARED((NPAD, D_HID), jnp.float32),
        pltpu.SemaphoreType.DMA,
        pltpu.SemaphoreType.DMA,
        pltpu.SemaphoreType.DMA,
        pltpu.SemaphoreType.DMA,
    ],
)


_BM = 512
_GRID = NPAD // _BM


def _dis_block(cnt_blk):
    deg = jnp.sum(cnt_blk, axis=0) + 1.0
    return lax.rsqrt(deg)


def _mm1_body(x_ref, w_ref, cnt_ref, o_ref):
    h = jnp.dot(x_ref[...], w_ref[...], preferred_element_type=jnp.float32,
                precision=lax.Precision.HIGHEST)
    o_ref[...] = h * _dis_block(cnt_ref[...])


_mm1 = pl.pallas_call(
    _mm1_body,
    grid=(_GRID,),
    in_specs=[
        pl.BlockSpec((_BM, D_IN), lambda i: (i, 0)),
        pl.BlockSpec((D_IN, D_HID), lambda i: (0, 0)),
        pl.BlockSpec((NW, _BM, 1), lambda i: (0, i, 0)),
    ],
    out_specs=pl.BlockSpec((_BM, D_HID), lambda i: (i, 0)),
    out_shape=jax.ShapeDtypeStruct((NPAD, D_HID), jnp.float32),
)


def _layer2_body(hp_ref, p_ref, cnt_ref, b1_ref, o_ref):
    dis = _dis_block(cnt_ref[...])
    acc = hp_ref[...] + p_ref[0] + p_ref[1]
    z = jnp.maximum(acc * dis + b1_ref[...], 0.0)
    o_ref[...] = z * dis


_layer2 = pl.pallas_call(
    _layer2_body,
    grid=(_GRID,),
    in_specs=[
        pl.BlockSpec((_BM, D_HID), lambda i: (i, 0)),
        pl.BlockSpec((NC, _BM, D_HID), lambda i: (0, i, 0)),
        pl.BlockSpec((NW, _BM, 1), lambda i: (0, i, 0)),
        pl.BlockSpec((1, D_HID), lambda i: (0, 0)),
    ],
    out_specs=pl.BlockSpec((_BM, D_HID), lambda i: (i, 0)),
    out_shape=jax.ShapeDtypeStruct((NPAD, D_HID), jnp.float32),
)

_BMF = 400
_GRIDF = N // _BMF


def _final_body(zp_ref, p_ref, cnt_ref, w2_ref, b2_ref, o_ref):
    dis = _dis_block(cnt_ref[...])
    agg = (zp_ref[...] + p_ref[0] + p_ref[1]) * dis
    y = jnp.dot(agg, w2_ref[...], preferred_element_type=jnp.float32,
                precision=lax.Precision.HIGHEST) + b2_ref[...]
    m = jnp.max(y, axis=1, keepdims=True)
    o_ref[...] = (y - m) - jnp.log(jnp.sum(jnp.exp(y - m), axis=1, keepdims=True))


_final = pl.pallas_call(
    _final_body,
    grid=(_GRIDF,),
    in_specs=[
        pl.BlockSpec((_BMF, D_HID), lambda i: (i, 0)),
        pl.BlockSpec((NC, _BMF, D_HID), lambda i: (0, i, 0)),
        pl.BlockSpec((NW, _BMF, 1), lambda i: (0, i, 0)),
        pl.BlockSpec((D_HID, D_OUT), lambda i: (0, 0)),
        pl.BlockSpec((1, D_OUT), lambda i: (0, 0)),
    ],
    out_specs=pl.BlockSpec((_BMF, D_OUT), lambda i: (i, 0)),
    out_shape=jax.ShapeDtypeStruct((N, D_OUT), jnp.float32),
)


def kernel(x, edge_index, W1, b1, W2, b2):
    src = edge_index[0].astype(jnp.int32)
    dst = edge_index[1].astype(jnp.int32)
    pad = jnp.full((EPAD - E,), N, jnp.int32)
    src_flat = jnp.concatenate([src, pad])
    dst_flat = jnp.concatenate([dst, pad])
    srci = jnp.sort(src_flat.reshape(NW, CH * K), axis=1).reshape(NW, CH, K)
    dsti = dst_flat.reshape(NW, CH, K)
    x_pad = jnp.zeros((NPAD, D_IN), jnp.float32).at[:N].set(x)
    zero128 = jnp.zeros((NPAD, D_HID), jnp.float32)

    cnts = _hist_call(dsti)
    cnts3 = cnts.reshape(NW, NPAD, 1)
    h1p = _mm1(x_pad, W1, cnts3)
    p1 = _scat(h1p, srci, dsti, zero128)
    z1p = _layer2(h1p, p1, cnts3, b1.reshape(1, D_HID))
    p2 = _scat(z1p, srci, dsti, zero128)
    return _final(z1p, p2, cnts3, W2, b2.reshape(1, D_OUT))


# speedup vs baseline: 3.0547x; 3.0547x over previous
import dataclasses

import jax
import jax.numpy as jnp
from jax import lax
from jax.experimental import pallas as pl
from jax.experimental.pallas import tpu as pltpu
from jax.experimental.pallas import tpu_sc as plsc

N = 10000
E = 320000
D_IN = 128
D_HID = 128
D_OUT = 16

NPAD = 10240
NC = 2
NS = 16
NW = NC * NS
L = 16
K = 128
CH = 80
KS = 32
NBUF = 4
EPAD = NW * CH * K

_mesh = plsc.VectorSubcoreMesh(core_axis_name="c", subcore_axis_name="s")


def _hist_body(dsti_hbm, out_hbm, dstv, cnt):
    c = lax.axis_index("c")
    s = lax.axis_index("s")
    wid = s * NC + c
    pltpu.sync_copy(dsti_hbm.at[wid], dstv)

    zeros = jnp.zeros((L,), jnp.float32)

    @pl.loop(0, NPAD // L, step=8)
    def _(i):
        for t in range(8):
            cnt[pl.ds((i + t) * L, L)] = zeros

    ones = jnp.ones((L,), jnp.float32)

    @pl.loop(0, CH)
    def _(j):
        for t in range(K // L):
            idx = dstv[j, pl.ds(t * L, L)]
            plsc.addupdate_scatter(cnt, [idx], ones)

    pltpu.sync_copy(cnt, out_hbm.at[wid])


_sc_params = pltpu.CompilerParams()
if "needs_layout_passes" in pltpu.CompilerParams.__dataclass_fields__:
    _sc_params = dataclasses.replace(_sc_params, needs_layout_passes=False)

_hist_call = pl.kernel(
    _hist_body,
    out_type=jax.ShapeDtypeStruct((NW, NPAD), jnp.float32),
    mesh=_mesh,
    compiler_params=_sc_params,
    scratch_types=[
        pltpu.VMEM((CH, K), jnp.int32),
        pltpu.VMEM((NPAD,), jnp.float32),
    ],
)


def _scat_body(table_hbm, srci_hbm, dsti_hbm, zero_hbm, out_hbm,
               srcv, dstv, b0, b1, b2, b3, acc, s0, s1, s2, s3):
    bufs = (b0, b1, b2, b3)
    sems = (s0, s1, s2, s3)
    c = lax.axis_index("c")
    s = lax.axis_index("s")
    wid = s * NC + c
    rpt = NPAD // NS
    base = s * rpt
    pltpu.sync_copy(zero_hbm.at[pl.ds(base, rpt)], acc.at[pl.ds(base, rpt)])
    pltpu.sync_copy(srci_hbm.at[wid], srcv)
    pltpu.sync_copy(dsti_hbm.at[wid], dstv)
    plsc.subcore_barrier()

    nq = K // KS
    nch = CH * nq

    def gath(chunk_row, off, q):
        return pltpu.make_async_copy(
            table_hbm.at[srcv.at[chunk_row, pl.ds(off, KS)]], bufs[q], sems[q])

    for q in range(NBUF - 1):
        gath(0, q * KS, q).start()

    @pl.loop(0, CH)
    def _(j):
        for q in range(NBUF):
            t = q + NBUF - 1
            row_adv, off = divmod(t, nq)

            @pl.when(nq * j + t < nch)
            def _():
                gath(j + row_adv, off * KS, (t % NBUF)).start()

            gath(j, q * KS, q).wait()
            pltpu.sync_copy(bufs[q], acc.at[dstv.at[j, pl.ds(q * KS, KS)]],
                            add=True)

    plsc.subcore_barrier()
    pltpu.sync_copy(acc.at[pl.ds(base, rpt)], out_hbm.at[c, pl.ds(base, rpt)])


_scat = pl.kernel(
    _scat_body,
    out_type=jax.ShapeDtypeStruct((NC, NPAD, D_HID), jnp.float32),
    mesh=_mesh,
    scratch_types=[
        pltpu.VMEM((CH, K), jnp.int32),
        pltpu.VMEM((CH, K), jnp.int32),
        pltpu.VMEM((KS, D_HID), jnp.float32),
        pltpu.VMEM((KS, D_HID), jnp.float32),
        pltpu.VMEM((KS, D_HID), jnp.float32),
        pltpu.VMEM((KS, D_HID), jnp.float32),
        pltpu.VMEM_SHARED((NPAD, D_HID), jnp.float32),
        pltpu.SemaphoreType.DMA,
        pltpu.SemaphoreType.DMA,
        pltpu.SemaphoreType.DMA,
        pltpu.SemaphoreType.DMA,
    ],
)


_BM = 512
_GRID = NPAD // _BM


def _dis_block(cnt_blk):
    deg = jnp.sum(cnt_blk, axis=0) + 1.0
    return lax.rsqrt(deg)


def _mm1_body(x_ref, w_ref, cnt_ref, o_ref):
    h = jnp.dot(x_ref[...], w_ref[...], preferred_element_type=jnp.float32,
                precision=lax.Precision.HIGHEST)
    o_ref[...] = h * _dis_block(cnt_ref[...])


_mm1 = pl.pallas_call(
    _mm1_body,
    grid=(_GRID,),
    in_specs=[
        pl.BlockSpec((_BM, D_IN), lambda i: (i, 0)),
        pl.BlockSpec((D_IN, D_HID), lambda i: (0, 0)),
        pl.BlockSpec((NW, _BM, 1), lambda i: (0, i, 0)),
    ],
    out_specs=pl.BlockSpec((_BM, D_HID), lambda i: (i, 0)),
    out_shape=jax.ShapeDtypeStruct((NPAD, D_HID), jnp.float32),
)


def _layer2_body(hp_ref, p_ref, cnt_ref, b1_ref, o_ref):
    dis = _dis_block(cnt_ref[...])
    acc = hp_ref[...] + p_ref[0] + p_ref[1]
    z = jnp.maximum(acc * dis + b1_ref[...], 0.0)
    o_ref[...] = z * dis


_layer2 = pl.pallas_call(
    _layer2_body,
    grid=(_GRID,),
    in_specs=[
        pl.BlockSpec((_BM, D_HID), lambda i: (i, 0)),
        pl.BlockSpec((NC, _BM, D_HID), lambda i: (0, i, 0)),
        pl.BlockSpec((NW, _BM, 1), lambda i: (0, i, 0)),
        pl.BlockSpec((1, D_HID), lambda i: (0, 0)),
    ],
    out_specs=pl.BlockSpec((_BM, D_HID), lambda i: (i, 0)),
    out_shape=jax.ShapeDtypeStruct((NPAD, D_HID), jnp.float32),
)

_BMF = 400
_GRIDF = N // _BMF


def _final_body(zp_ref, p_ref, cnt_ref, w2_ref, b2_ref, o_ref):
    dis = _dis_block(cnt_ref[...])
    agg = (zp_ref[...] + p_ref[0] + p_ref[1]) * dis
    y = jnp.dot(agg, w2_ref[...], preferred_element_type=jnp.float32,
                precision=lax.Precision.HIGHEST) + b2_ref[...]
    m = jnp.max(y, axis=1, keepdims=True)
    o_ref[...] = (y - m) - jnp.log(jnp.sum(jnp.exp(y - m), axis=1, keepdims=True))


_final = pl.pallas_call(
    _final_body,
    grid=(_GRIDF,),
    in_specs=[
        pl.BlockSpec((_BMF, D_HID), lambda i: (i, 0)),
        pl.BlockSpec((NC, _BMF, D_HID), lambda i: (0, i, 0)),
        pl.BlockSpec((NW, _BMF, 1), lambda i: (0, i, 0)),
        pl.BlockSpec((D_HID, D_OUT), lambda i: (0, 0)),
        pl.BlockSpec((1, D_OUT), lambda i: (0, 0)),
    ],
    out_specs=pl.BlockSpec((_BMF, D_OUT), lambda i: (i, 0)),
    out_shape=jax.ShapeDtypeStruct((N, D_OUT), jnp.float32),
)


def kernel(x, edge_index, W1, b1, W2, b2):
    src = edge_index[0].astype(jnp.int32)
    dst = edge_index[1].astype(jnp.int32)
    pad = N + (jnp.arange(EPAD - E, dtype=jnp.int32) % (NPAD - N))
    src_flat = jnp.concatenate([src, pad])
    dst_flat = jnp.concatenate([dst, pad])
    srci = src_flat.reshape(NW, CH, K)
    dsti = dst_flat.reshape(NW, CH, K)
    x_pad = jnp.zeros((NPAD, D_IN), jnp.float32).at[:N].set(x)
    zero128 = jnp.zeros((NPAD, D_HID), jnp.float32)

    cnts = _hist_call(dsti)
    cnts3 = cnts.reshape(NW, NPAD, 1)
    h1p = _mm1(x_pad, W1, cnts3)
    p1 = _scat(h1p, srci, dsti, zero128)
    z1p = _layer2(h1p, p1, cnts3, b1.reshape(1, D_HID))
    p2 = _scat(z1p, srci, dsti, zero128)
    return _final(z1p, p2, cnts3, W2, b2.reshape(1, D_OUT))


# speedup vs baseline: 4.7147x; 1.5434x over previous
import dataclasses

import jax
import jax.numpy as jnp
from jax import lax
from jax.experimental import pallas as pl
from jax.experimental.pallas import tpu as pltpu
from jax.experimental.pallas import tpu_sc as plsc

N = 10000
E = 320000
D_IN = 128
D_HID = 128
D_OUT = 16

NPAD = 10240
NC = 2
NS = 16
NW = NC * NS
L = 16
K = 128
CH = 80
KS = 32
NBUF = 4
EPAD = NW * CH * K

_mesh = plsc.VectorSubcoreMesh(core_axis_name="c", subcore_axis_name="s")


def _hist_body(dsti_hbm, out_hbm, dstv, cnt):
    c = lax.axis_index("c")
    s = lax.axis_index("s")
    wid = s * NC + c
    pltpu.sync_copy(dsti_hbm.at[wid], dstv)

    zeros = jnp.zeros((L,), jnp.float32)

    @pl.loop(0, NPAD // L, step=8)
    def _(i):
        for t in range(8):
            cnt[pl.ds((i + t) * L, L)] = zeros

    ones = jnp.ones((L,), jnp.float32)

    @pl.loop(0, CH)
    def _(j):
        for t in range(K // L):
            idx = dstv[j, pl.ds(t * L, L)]
            plsc.addupdate_scatter(cnt, [idx], ones)

    pltpu.sync_copy(cnt, out_hbm.at[wid])


_sc_params = pltpu.CompilerParams()
if "needs_layout_passes" in pltpu.CompilerParams.__dataclass_fields__:
    _sc_params = dataclasses.replace(_sc_params, needs_layout_passes=False)

_hist_call = pl.kernel(
    _hist_body,
    out_type=jax.ShapeDtypeStruct((NW, NPAD), jnp.float32),
    mesh=_mesh,
    compiler_params=_sc_params,
    scratch_types=[
        pltpu.VMEM((CH, K), jnp.int32),
        pltpu.VMEM((NPAD,), jnp.float32),
    ],
)


def _scat_body(table_hbm, srci_hbm, dsti_hbm, zero_hbm, out_hbm,
               srcv, dstv, b0, b1, b2, b3, acc, s0, s1, s2, s3):
    bufs = (b0, b1, b2, b3)
    sems = (s0, s1, s2, s3)
    c = lax.axis_index("c")
    s = lax.axis_index("s")
    wid = s * NC + c
    rpt = NPAD // NS
    base = s * rpt
    pltpu.sync_copy(zero_hbm.at[pl.ds(base, rpt)], acc.at[pl.ds(base, rpt)])
    pltpu.sync_copy(srci_hbm.at[wid], srcv)
    pltpu.sync_copy(dsti_hbm.at[wid], dstv)
    plsc.subcore_barrier()

    nq = K // KS
    nch = CH * nq

    def gath(chunk_row, off, q):
        return pltpu.make_async_copy(
            table_hbm.at[srcv.at[chunk_row, pl.ds(off, KS)]], bufs[q], sems[q])

    for q in range(NBUF - 1):
        gath(0, q * KS, q).start()

    @pl.loop(0, CH)
    def _(j):
        for q in range(NBUF):
            t = q + NBUF - 1
            row_adv, off = divmod(t, nq)

            @pl.when(nq * j + t < nch)
            def _():
                gath(j + row_adv, off * KS, (t % NBUF)).start()

            gath(j, q * KS, q).wait()
            pltpu.sync_copy(bufs[q], acc.at[dstv.at[j, pl.ds(q * KS, KS)]],
                            add=True)

    plsc.subcore_barrier()
    pltpu.sync_copy(acc.at[pl.ds(base, rpt)], out_hbm.at[c, pl.ds(base, rpt)])


_scat = pl.kernel(
    _scat_body,
    out_type=jax.ShapeDtypeStruct((NC, NPAD, D_HID), jnp.float32),
    mesh=_mesh,
    scratch_types=[
        pltpu.VMEM((CH, K), jnp.int32),
        pltpu.VMEM((CH, K), jnp.int32),
        pltpu.VMEM((KS, D_HID), jnp.float32),
        pltpu.VMEM((KS, D_HID), jnp.float32),
        pltpu.VMEM((KS, D_HID), jnp.float32),
        pltpu.VMEM((KS, D_HID), jnp.float32),
        pltpu.VMEM_SHARED((NPAD, D_HID), jnp.float32),
        pltpu.SemaphoreType.DMA,
        pltpu.SemaphoreType.DMA,
        pltpu.SemaphoreType.DMA,
        pltpu.SemaphoreType.DMA,
    ],
)


_BM = 400
_GRIDM = N // _BM


def _mm1_body(x_ref, w_ref, cnt_ref, o_ref, dis_ref):
    deg = jnp.sum(cnt_ref[...], axis=1, keepdims=True) + 1.0
    dis = lax.rsqrt(deg)
    dis_ref[...] = dis
    h = jnp.dot(x_ref[...], w_ref[...], preferred_element_type=jnp.float32)
    o_ref[...] = h * dis


_mm1 = pl.pallas_call(
    _mm1_body,
    grid=(_GRIDM,),
    in_specs=[
        pl.BlockSpec((_BM, D_IN), lambda i: (i, 0)),
        pl.BlockSpec((D_IN, D_HID), lambda i: (0, 0)),
        pl.BlockSpec((_BM, NW), lambda i: (i, 0)),
    ],
    out_specs=[
        pl.BlockSpec((_BM, D_HID), lambda i: (i, 0)),
        pl.BlockSpec((_BM, 1), lambda i: (i, 0)),
    ],
    out_shape=[
        jax.ShapeDtypeStruct((NPAD, D_HID), jnp.float32),
        jax.ShapeDtypeStruct((NPAD, 1), jnp.float32),
    ],
)


def _layer2_body(hp_ref, p_ref, dis_ref, b1_ref, o_ref):
    dis = dis_ref[...]
    acc = hp_ref[...] + p_ref[0] + p_ref[1]
    z = jnp.maximum(acc * dis + b1_ref[...], 0.0)
    o_ref[...] = z * dis


_layer2 = pl.pallas_call(
    _layer2_body,
    grid=(_GRIDM,),
    in_specs=[
        pl.BlockSpec((_BM, D_HID), lambda i: (i, 0)),
        pl.BlockSpec((NC, _BM, D_HID), lambda i: (0, i, 0)),
        pl.BlockSpec((_BM, 1), lambda i: (i, 0)),
        pl.BlockSpec((1, D_HID), lambda i: (0, 0)),
    ],
    out_specs=pl.BlockSpec((_BM, D_HID), lambda i: (i, 0)),
    out_shape=jax.ShapeDtypeStruct((NPAD, D_HID), jnp.float32),
)


def _final_body(zp_ref, p_ref, dis_ref, w2_ref, b2_ref, o_ref):
    dis = dis_ref[...]
    agg = (zp_ref[...] + p_ref[0] + p_ref[1]) * dis
    y = jnp.dot(agg, w2_ref[...], preferred_element_type=jnp.float32) + b2_ref[...]
    m = jnp.max(y, axis=1, keepdims=True)
    o_ref[...] = (y - m) - jnp.log(jnp.sum(jnp.exp(y - m), axis=1, keepdims=True))


_final = pl.pallas_call(
    _final_body,
    grid=(_GRIDM,),
    in_specs=[
        pl.BlockSpec((_BM, D_HID), lambda i: (i, 0)),
        pl.BlockSpec((NC, _BM, D_HID), lambda i: (0, i, 0)),
        pl.BlockSpec((_BM, 1), lambda i: (i, 0)),
        pl.BlockSpec((D_HID, D_OUT), lambda i: (0, 0)),
        pl.BlockSpec((1, D_OUT), lambda i: (0, 0)),
    ],
    out_specs=pl.BlockSpec((_BM, D_OUT), lambda i: (i, 0)),
    out_shape=jax.ShapeDtypeStruct((N, D_OUT), jnp.float32),
)


def kernel(x, edge_index, W1, b1, W2, b2):
    src = edge_index[0].astype(jnp.int32)
    dst = edge_index[1].astype(jnp.int32)
    pad = N + (jnp.arange(EPAD - E, dtype=jnp.int32) % (NPAD - N))
    src_flat = jnp.concatenate([src, pad])
    dst_flat = jnp.concatenate([dst, pad])
    srci = src_flat.reshape(NW, CH, K)
    dsti = dst_flat.reshape(NW, CH, K)
    zero128 = jnp.zeros((NPAD, D_HID), jnp.float32)

    cnts = _hist_call(dsti)
    cntsT = cnts.T
    h1p, dis = _mm1(x, W1, cntsT)
    p1 = _scat(h1p, srci, dsti, zero128)
    z1p = _layer2(h1p, p1, dis, b1.reshape(1, D_HID))
    p2 = _scat(z1p, srci, dsti, zero128)
    return _final(z1p, p2, dis, W2, b2.reshape(1, D_OUT))
